# baseline (speedup 1.0000x reference)
import functools

import jax
import jax.numpy as jnp
from jax import lax
from jax.experimental import pallas as pl
from jax.experimental.pallas import tpu as pltpu
from jax.experimental.pallas import tpu_sc as plsc

_NC = 2
_NS = 16
_LANES = 16


def _build_kernel(N, V, D, NG, C):
    NW = _NC * _NS
    rows_per_w = N // NW
    n_chunks = rows_per_w // C
    RPB = 4
    SPB = 128 // NG

    mesh = plsc.VectorSubcoreMesh(
        core_axis_name="c", subcore_axis_name="s",
        num_cores=_NC, num_subcores=_NS)

    buf = lambda shape, dtype: [pltpu.VMEM(shape, dtype) for _ in range(2)]

    @functools.partial(
        pl.kernel,
        out_type=jax.ShapeDtypeStruct((N * D,), jnp.float32),
        mesh=mesh,
        compiler_params=pltpu.CompilerParams(needs_layout_passes=False),
        scratch_types=[
            pltpu.VMEM((rows_per_w + _LANES,), jnp.int32),
            buf((C,), jnp.int32),
            buf((C,), jnp.int32),
            buf((C, 128), jnp.int32),
            buf((C, 128), jnp.float32),
            buf((C * D,), jnp.float32),
            [pltpu.SemaphoreType.DMA for _ in range(2)],
            [pltpu.SemaphoreType.DMA for _ in range(2)],
            [pltpu.SemaphoreType.DMA for _ in range(2)],
        ],
    )
    def dequant(idx_hbm, w8_hbm, scl_hbm, out_hbm,
                idx_v, bidx_v, sidx_v, rows_v, sblk_v, out_v,
                gsem, ssem, osem):
        wblk_hbm = w8_hbm.bitcast(jnp.int32).reshape(V * D // 4 // 128, 128)
        sblk_hbm = scl_hbm
        wid = lax.axis_index("s") * _NC + lax.axis_index("c")
        base = wid * rows_per_w
        pltpu.sync_copy(idx_hbm.at[pl.ds(base, rows_per_w)],
                        idx_v.at[pl.ds(0, rows_per_w)])

        def start_gathers(c, b):
            cbase = c * C

            @plsc.parallel_loop(0, C // _LANES, unroll=4)
            def build_idx(t):
                iv = idx_v[pl.ds(cbase + t * _LANES, _LANES)]
                bidx_v[b][pl.ds(t * _LANES, _LANES)] = iv >> 2
                sidx_v[b][pl.ds(t * _LANES, _LANES)] = iv >> 5

            pltpu.async_copy(wblk_hbm.at[bidx_v[b]], rows_v[b], gsem[b])
            pltpu.async_copy(sblk_hbm.at[sidx_v[b]], sblk_v[b], ssem[b])

        def compute_chunk(c, b):
            cbase = c * C

            @plsc.parallel_loop(0, C, unroll=4)
            def do_row(r):
                ivv = idx_v[pl.ds(cbase + r, _LANES)]
                iv0 = ivv[0]
                shl = 24 - 8 * (iv0 & (RPB - 1))
                off = (iv0 & (SPB - 1)) * NG
                cl = jnp.minimum(off, 128 - _LANES)
                delta = off - cl
                sv = sblk_v[b][r, pl.ds(cl, _LANES)]
                for t in range(8):
                    s = jnp.take(
                        sv, jnp.full((_LANES,), t // 2, jnp.int32) + delta,
                        axis=0)
                    w = rows_v[b][r, pl.ds(t * _LANES, _LANES)]
                    v = (w << shl) >> 24
                    y = v.astype(jnp.float32) * s
                    out_v[b][pl.ds(r * D + t * _LANES, _LANES)] = y

        start_gathers(0, 0)

        def do_pair(g2, _):
            for b in range(2):
                c = g2 * 2 + b
                pltpu.make_async_copy(
                    wblk_hbm.at[bidx_v[b]], rows_v[b], gsem[b]).wait()
                pltpu.make_async_copy(
                    sblk_hbm.at[sidx_v[b]], sblk_v[b], ssem[b]).wait()

                @pl.when(c + 1 < n_chunks)
                def _():
                    start_gathers(c + 1, 1 - b)

                @pl.when(c >= 2)
                def _():
                    pltpu.make_async_copy(
                        out_v[b],
                        out_hbm.at[pl.ds((base + (c - 2) * C) * D, C * D)],
                        osem[b]).wait()

                compute_chunk(c, b)
                pltpu.async_copy(
                    out_v[b], out_hbm.at[pl.ds((base + c * C) * D, C * D)],
                    osem[b])
            return 0

        lax.fori_loop(0, n_chunks // 2, do_pair, 0, unroll=False)

        for b in range(2):
            c = n_chunks - 2 + b
            pltpu.make_async_copy(
                out_v[b], out_hbm.at[pl.ds((base + c * C) * D, C * D)],
                osem[b]).wait()

    return dequant


@jax.jit
def kernel(indices, weight, scales):
    B, L = indices.shape
    V, D = weight.shape
    NG = scales.shape[1]
    N = B * L
    idx_flat = indices.reshape(N)
    scl_blocks = scales.reshape(V * NG // 128, 128)
    fn = _build_kernel(N, V, D, NG, C=128)
    out = fn(idx_flat, weight, scl_blocks)
    return out.reshape(B, L, D)


# speedup vs baseline: 1.0618x; 1.0618x over previous
import functools

import jax
import jax.numpy as jnp
from jax import lax
from jax.experimental import pallas as pl
from jax.experimental.pallas import tpu as pltpu
from jax.experimental.pallas import tpu_sc as plsc

_NC = 2
_NS = 16
_LANES = 16


def _build_kernel(B, L, V, D, NG, CB):
    NW = _NC * _NS
    bpt = B // NW
    rows_per_w = bpt * L
    n_chunks = bpt // CB
    C = CB * L
    CPAD = -(-C // _LANES) * _LANES
    RPB = 4
    SPB = 128 // NG

    mesh = plsc.VectorSubcoreMesh(
        core_axis_name="c", subcore_axis_name="s",
        num_cores=_NC, num_subcores=_NS)

    buf = lambda shape, dtype: [pltpu.VMEM(shape, dtype) for _ in range(2)]

    @functools.partial(
        pl.kernel,
        out_type=jax.ShapeDtypeStruct((B, L, D), jnp.float32),
        mesh=mesh,
        compiler_params=pltpu.CompilerParams(needs_layout_passes=False),
        scratch_types=[
            pltpu.VMEM((rows_per_w + CPAD,), jnp.int32),
            buf((CPAD,), jnp.int32),
            buf((CPAD,), jnp.int32),
            buf((CPAD, 128), jnp.int32),
            buf((CPAD, 128), jnp.float32),
            buf((C, D), jnp.float32),
            [pltpu.SemaphoreType.DMA for _ in range(2)],
            [pltpu.SemaphoreType.DMA for _ in range(2)],
            [pltpu.SemaphoreType.DMA for _ in range(2 * CB)],
        ],
    )
    def dequant(idx_hbm, w8_hbm, scl_hbm, out_hbm,
                idx_v, bidx_v, sidx_v, rows_v, sblk_v, out_v,
                gsem, ssem, osem):
        wblk_hbm = w8_hbm.bitcast(jnp.int32).reshape(V * D // 4 // 128, 128)
        wid = lax.axis_index("s") * _NC + lax.axis_index("c")
        base = wid * rows_per_w
        b0 = wid * bpt
        pltpu.sync_copy(idx_hbm.at[pl.ds(base, rows_per_w)],
                        idx_v.at[pl.ds(0, rows_per_w)])
        for t in range(CPAD // _LANES):
            idx_v[pl.ds(rows_per_w + t * _LANES, _LANES)] = jnp.zeros(
                (_LANES,), jnp.int32)

        def start_gathers(c, b):
            cbase = c * C

            @plsc.parallel_loop(0, CPAD // _LANES, unroll=4)
            def build_idx(t):
                iv = idx_v[pl.ds(cbase + t * _LANES, _LANES)]
                bidx_v[b][pl.ds(t * _LANES, _LANES)] = iv >> 2
                sidx_v[b][pl.ds(t * _LANES, _LANES)] = iv >> 5

            pltpu.async_copy(wblk_hbm.at[bidx_v[b]], rows_v[b], gsem[b])
            pltpu.async_copy(scl_hbm.at[sidx_v[b]], sblk_v[b], ssem[b])

        def compute_chunk(c, b):
            cbase = c * C

            @plsc.parallel_loop(0, C, unroll=4)
            def do_row(r):
                ivv = idx_v[pl.ds(cbase + r, _LANES)]
                iv0 = ivv[0]
                shl = 24 - 8 * (iv0 & (RPB - 1))
                off = (iv0 & (SPB - 1)) * NG
                cl = jnp.minimum(off, 128 - _LANES)
                delta = off - cl
                sv = sblk_v[b][r, pl.ds(cl, _LANES)]
                for t in range(8):
                    s = jnp.take(
                        sv, jnp.full((_LANES,), t // 2, jnp.int32) + delta,
                        axis=0)
                    w = rows_v[b][r, pl.ds(t * _LANES, _LANES)]
                    v = (w << shl) >> 24
                    y = v.astype(jnp.float32) * s
                    out_v[b][r, pl.ds(t * _LANES, _LANES)] = y

        start_gathers(0, 0)

        def do_pair(g2, _):
            for b in range(2):
                c = g2 * 2 + b
                pltpu.make_async_copy(
                    wblk_hbm.at[bidx_v[b]], rows_v[b], gsem[b]).wait()
                pltpu.make_async_copy(
                    scl_hbm.at[sidx_v[b]], sblk_v[b], ssem[b]).wait()

                @pl.when(c + 1 < n_chunks)
                def _():
                    start_gathers(c + 1, 1 - b)

                @pl.when(c >= 2)
                def _():
                    for k in range(CB):
                        pltpu.make_async_copy(
                            out_v[b].at[pl.ds(k * L, L)],
                            out_hbm.at[b0 + (c - 2) * CB + k],
                            osem[b * CB + k]).wait()

                compute_chunk(c, b)
                for k in range(CB):
                    pltpu.async_copy(
                        out_v[b].at[pl.ds(k * L, L)],
                        out_hbm.at[b0 + c * CB + k], osem[b * CB + k])
            return 0

        lax.fori_loop(0, n_chunks // 2, do_pair, 0, unroll=False)

        for b in range(2):
            c = n_chunks - 2 + b
            for k in range(CB):
                pltpu.make_async_copy(
                    out_v[b].at[pl.ds(k * L, L)],
                    out_hbm.at[b0 + c * CB + k], osem[b * CB + k]).wait()

    return dequant


@jax.jit
def kernel(indices, weight, scales):
    B, L = indices.shape
    V, D = weight.shape
    NG = scales.shape[1]
    N = B * L
    idx_flat = indices.reshape(N)
    scl_blocks = scales.reshape(V * NG // 128, 128)
    fn = _build_kernel(B, L, V, D, NG, CB=2)
    return fn(idx_flat, weight, scl_blocks)
